# baseline (speedup 1.0000x reference)
import functools

import jax
import jax.numpy as jnp
from jax import lax
from jax.experimental import pallas as pl
from jax.experimental.pallas import tpu as pltpu
from jax.experimental.pallas import tpu_sc as plsc

DIM = 64
PADDIM = 128
NC = 2
NS = 16
NW = NC * NS
CHUNK = 256
SUPER = 1


@functools.partial(jax.jit, static_argnames=("total",))
def _sc_gather(values, table, total):
    b_per_w = total // NW
    sup = SUPER * CHUNK
    n_super = b_per_w // sup
    pad_mat = jnp.concatenate(
        [jnp.eye(DIM, dtype=jnp.float32),
         jnp.zeros((DIM, PADDIM - DIM), jnp.float32)], axis=1)
    table128 = jax.lax.dot(
        table, pad_mat, precision=jax.lax.Precision.HIGHEST)
    mesh = plsc.VectorSubcoreMesh(core_axis_name="c", subcore_axis_name="s")

    @functools.partial(
        pl.kernel,
        out_type=jax.ShapeDtypeStruct((total, PADDIM), jnp.float32),
        mesh=mesh,
        scratch_types=[
            pltpu.VMEM((b_per_w,), jnp.int32),
            [pltpu.VMEM((sup, PADDIM), jnp.float32) for _ in range(2)],
            [pltpu.SemaphoreType.DMA for _ in range(2)],
            [pltpu.SemaphoreType.DMA for _ in range(2)],
        ],
    )
    def k(idx_hbm, table_hbm, out_hbm, idx_v, rows, gsem, wsem):
        wid = lax.axis_index("s") * NC + lax.axis_index("c")
        base = wid * b_per_w
        pltpu.sync_copy(idx_hbm.at[pl.ds(base, b_per_w)], idx_v)

        def fire(s, b):
            for j in range(SUPER):
                pltpu.async_copy(
                    table_hbm.at[idx_v.at[pl.ds((s * SUPER + j) * CHUNK, CHUNK)]],
                    rows[b].at[pl.ds(j * CHUNK, CHUNK)],
                    gsem[b],
                )

        def drain(b):
            for j in range(SUPER):
                pltpu.make_async_copy(
                    table_hbm.at[idx_v.at[pl.ds(j * CHUNK, CHUNK)]],
                    rows[b].at[pl.ds(j * CHUNK, CHUNK)],
                    gsem[b],
                ).wait()

        fire(0, 0)

        @pl.loop(0, n_super, step=2)
        def _(s):
            for b in range(2):
                g = s + b
                nb = 1 - b
                @pl.when(g + 1 < n_super)
                def _():
                    @pl.when(g >= 1)
                    def _():
                        pltpu.make_async_copy(
                            rows[nb],
                            out_hbm.at[pl.ds(base, sup)],
                            wsem[nb],
                        ).wait()
                    fire(g + 1, nb)

                drain(b)
                pltpu.async_copy(
                    rows[b], out_hbm.at[pl.ds(base + g * sup, sup)], wsem[b]
                )

        pltpu.make_async_copy(rows[0], out_hbm.at[pl.ds(base, sup)], wsem[0]).wait()
        pltpu.make_async_copy(rows[1], out_hbm.at[pl.ds(base, sup)], wsem[1]).wait()

    return k(values, table128)[:, :DIM]


def kernel(values, offsets, table):
    del offsets
    total = values.shape[0]
    return _sc_gather(values.astype(jnp.int32), table, total)


# speedup vs baseline: 1.3828x; 1.3828x over previous
import functools

import jax
import jax.numpy as jnp
from jax import lax
from jax.experimental import pallas as pl
from jax.experimental.pallas import tpu as pltpu
from jax.experimental.pallas import tpu_sc as plsc

DIM = 64
PADDIM = 128
NC = 2
NS = 16
NW = NC * NS
CHUNK = 256
SUPER = 1


@functools.partial(jax.jit, static_argnames=("total",))
def _sc_gather(values, table, total):
    b_per_w = total // NW
    sup = SUPER * CHUNK
    n_super = b_per_w // sup
    table128 = jnp.concatenate(
        [table, jnp.zeros((table.shape[0], PADDIM - DIM), jnp.float32)], axis=1)
    mesh = plsc.VectorSubcoreMesh(core_axis_name="c", subcore_axis_name="s")

    @functools.partial(
        pl.kernel,
        out_type=jax.ShapeDtypeStruct((total, PADDIM), jnp.float32),
        mesh=mesh,
        scratch_types=[
            pltpu.VMEM((b_per_w,), jnp.int32),
            [pltpu.VMEM((sup, PADDIM), jnp.float32) for _ in range(2)],
            [pltpu.SemaphoreType.DMA for _ in range(2)],
            [pltpu.SemaphoreType.DMA for _ in range(2)],
        ],
    )
    def k(idx_hbm, table_hbm, out_hbm, idx_v, rows, gsem, wsem):
        wid = lax.axis_index("s") * NC + lax.axis_index("c")
        base = wid * b_per_w
        pltpu.sync_copy(idx_hbm.at[pl.ds(base, b_per_w)], idx_v)

        def fire(s, b):
            for j in range(SUPER):
                pltpu.async_copy(
                    table_hbm.at[idx_v.at[pl.ds((s * SUPER + j) * CHUNK, CHUNK)]],
                    rows[b].at[pl.ds(j * CHUNK, CHUNK)],
                    gsem[b],
                )

        def drain(b):
            for j in range(SUPER):
                pltpu.make_async_copy(
                    table_hbm.at[idx_v.at[pl.ds(j * CHUNK, CHUNK)]],
                    rows[b].at[pl.ds(j * CHUNK, CHUNK)],
                    gsem[b],
                ).wait()

        fire(0, 0)

        @pl.loop(0, n_super, step=2)
        def _(s):
            for b in range(2):
                g = s + b
                nb = 1 - b
                @pl.when(g + 1 < n_super)
                def _():
                    @pl.when(g >= 1)
                    def _():
                        pltpu.make_async_copy(
                            rows[nb],
                            out_hbm.at[pl.ds(base, sup)],
                            wsem[nb],
                        ).wait()
                    fire(g + 1, nb)

                drain(b)
                pltpu.async_copy(
                    rows[b], out_hbm.at[pl.ds(base + g * sup, sup)], wsem[b]
                )

        pltpu.make_async_copy(rows[0], out_hbm.at[pl.ds(base, sup)], wsem[0]).wait()
        pltpu.make_async_copy(rows[1], out_hbm.at[pl.ds(base, sup)], wsem[1]).wait()

    return k(values, table128)[:, :DIM]


def kernel(values, offsets, table):
    del offsets
    total = values.shape[0]
    return _sc_gather(values.astype(jnp.int32), table, total)
